# baseline (speedup 1.0000x reference)
import functools

import jax
import jax.numpy as jnp
from jax import lax
from jax.experimental import pallas as pl
from jax.experimental.pallas import tpu as pltpu
from jax.experimental.pallas import tpu_sc as plsc

B_ROWS = 4096
SEQ = 200
EMBED = 64

NC = 2
NS = 16
NW = NC * NS
BBLK = B_ROWS // NW
L = 16


@functools.partial(
    pl.kernel,
    mesh=plsc.VectorSubcoreMesh(core_axis_name="c", subcore_axis_name="s"),
    compiler_params=pltpu.CompilerParams(use_tc_tiling_on_sc=False),
    out_type=jax.ShapeDtypeStruct((B_ROWS, SEQ, EMBED), jnp.float32),
    scratch_types=(
        [pltpu.VMEM((SEQ, BBLK), jnp.int32)]
        + [pltpu.VMEM((BBLK, EMBED), jnp.float32) for _ in range(2)]
        + [pltpu.SemaphoreType.DMA for _ in range(4)]
    ),
)
def _emb_lookup(xt_hbm, table_hbm, out_hbm, idx_v, *rest):
    rows = rest[0:2]
    gsem = rest[2:4]
    ssem = rest[4:6]
    wid = lax.axis_index("s") * NC + lax.axis_index("c")
    b0 = wid * BBLK

    pltpu.sync_copy(xt_hbm.at[:, pl.ds(b0, BBLK)], idx_v)

    def start_gather(t, slot):
        pltpu.async_copy(table_hbm.at[idx_v.at[t]], rows[slot], gsem[slot])

    def wait_gather(t, slot):
        pltpu.make_async_copy(table_hbm.at[idx_v.at[t]], rows[slot], gsem[slot]).wait()

    def start_store(t, slot):
        pltpu.async_copy(
            rows[slot], out_hbm.at[pl.ds(b0, BBLK), t], ssem[slot]
        )

    def wait_store(t, slot):
        pltpu.make_async_copy(
            rows[slot], out_hbm.at[pl.ds(b0, BBLK), t], ssem[slot]
        ).wait()

    start_gather(0, 0)
    start_gather(1, 1)

    def pair_body(p, carry):
        for slot in (0, 1):
            t = p * 2 + slot
            wait_gather(t, slot)

            @pl.when(p > 0)
            def _(slot=slot, t=t):
                wait_store(t - 2, slot)

            start_store(t, slot)

            @pl.when(t + 2 < SEQ)
            def _(slot=slot, t=t):
                start_gather(t + 2, slot)

        return carry

    lax.fori_loop(0, SEQ // 2, pair_body, 0)
    wait_store(SEQ - 2, 0)
    wait_store(SEQ - 1, 1)


def kernel(x, table):
    xt = x.T.astype(jnp.int32)
    return _emb_lookup(xt, table)
